# Initial kernel scaffold; baseline (speedup 1.0000x reference)
import jax
import jax.numpy as jnp
from jax.experimental import pallas as pl


def kernel(x, edge_index, W1, b1, W2, b2, W3, b3):
    raise NotImplementedError("write your pallas kernel here")


# speedup vs baseline: 10.5695x; 10.5695x over previous
import functools

import jax
import jax.numpy as jnp
from jax import lax
from jax.experimental import pallas as pl
from jax.experimental.pallas import tpu as pltpu
from jax.experimental.pallas import tpu_sc as plsc

N = 100000
E = 1600000
NACC = 102400
EP = 1601536
NCORE = 2
NSUB = 16
BATCH = 128
SLICE = NACC // NSUB
ZROWS = 400
PW_ES = EP // (NCORE * NSUB)
NB_ES = PW_ES // BATCH
PW_FS = EP // NSUB
NB_FS = PW_FS // BATCH

_MESH = plsc.VectorSubcoreMesh(
    core_axis_name="c", subcore_axis_name="s", num_cores=NCORE, num_subcores=NSUB
)
_SC_PARAMS = pltpu.CompilerParams(use_tc_tiling_on_sc=False)


def _zero_rows16(zb):
  def body(i, _):
    zb[i, :] = jnp.zeros((16,), jnp.float32)
    return 0
  lax.fori_loop(0, ZROWS, body, 0, unroll=False)


def _zero_acc_slice16(acc, zb, rowbase):
  for r in range(SLICE // ZROWS):
    pltpu.sync_copy(zb, acc.at[pl.ds(rowbase + r * ZROWS, ZROWS)])


@functools.partial(
    pl.kernel,
    out_type=[
        jax.ShapeDtypeStruct((NACC,), jnp.float32),
        jax.ShapeDtypeStruct((NACC,), jnp.float32),
    ],
    mesh=_MESH,
    compiler_params=_SC_PARAMS,
    scratch_types=[
        pltpu.VMEM_SHARED((NACC,), jnp.float32),
        pltpu.VMEM((1, BATCH), jnp.int32),
        pltpu.VMEM((1, BATCH), jnp.float32),
        pltpu.VMEM((SLICE,), jnp.float32),
    ],
)
def _sc_degree(dst_h, out0_h, out1_h, acc, db, ones, zb):
  c = lax.axis_index("c")
  s = lax.axis_index("s")

  def zbody(i, _):
    zb[pl.ds(i * 16, 16)] = jnp.zeros((16,), jnp.float32)
    return 0
  lax.fori_loop(0, SLICE // 16, zbody, 0, unroll=False)
  for k in range(BATCH // 16):
    ones[0, pl.ds(k * 16, 16)] = jnp.full((16,), 1.0, jnp.float32)

  rowbase = s * SLICE
  pltpu.sync_copy(zb, acc.at[pl.ds(rowbase, SLICE)])
  plsc.subcore_barrier()

  ebase = (c * NSUB + s) * PW_ES

  def ebody(j, _):
    base = ebase + j * BATCH
    pltpu.sync_copy(dst_h.at[pl.ds(base, BATCH)], db.at[0])
    pltpu.sync_copy(ones.at[0], acc.at[db.at[0]], add=True)
    return 0
  lax.fori_loop(0, NB_ES, ebody, 0, unroll=False)

  plsc.subcore_barrier()
  pltpu.sync_copy(acc.at[pl.ds(rowbase, SLICE)], zb)
  for k, out_h in enumerate((out0_h, out1_h)):
    @pl.when(c == k)
    def _():
      pltpu.sync_copy(zb, out_h.at[pl.ds(rowbase, SLICE)])


def _make_agg16(feature_split):
  n_out = 4 if feature_split else NCORE

  @functools.partial(
      pl.kernel,
      out_type=[jax.ShapeDtypeStruct((NACC, 16), jnp.float32)] * n_out,
      mesh=_MESH,
      compiler_params=_SC_PARAMS,
      scratch_types=[
          pltpu.VMEM_SHARED((NACC, 16), jnp.float32),
          pltpu.VMEM((1, BATCH), jnp.int32),
          pltpu.VMEM((1, BATCH), jnp.int32),
          pltpu.VMEM((1, BATCH), jnp.int32),
          pltpu.VMEM((BATCH, 16), jnp.float32),
          pltpu.VMEM((ZROWS, 16), jnp.float32),
          pltpu.VMEM((ZROWS, 16), jnp.float32),
          pltpu.SemaphoreType.DMA,
      ],
  )
  def agg(table_h, src_h, dst_h, *rest):
    outs = rest[:n_out]
    acc, sb, db, gb, rows, zb, cb, sem = rest[n_out:]
    c = lax.axis_index("c")
    s = lax.axis_index("s")
    _zero_rows16(zb)
    rowbase = s * SLICE

    for fc in range(2 if feature_split else 1):
      if feature_split:
        cid = c * 2 + fc
        ebase = s * PW_FS
        nb = NB_FS
      else:
        cid = c
        ebase = (c * NSUB + s) * PW_ES
        nb = NB_ES

      _zero_acc_slice16(acc, zb, rowbase)
      plsc.subcore_barrier()

      def ebody(j, _):
        base = ebase + j * BATCH
        pltpu.sync_copy(src_h.at[pl.ds(base, BATCH)], sb.at[0])
        pltpu.sync_copy(dst_h.at[pl.ds(base, BATCH)], db.at[0])
        if feature_split:
          for k in range(BATCH // 16):
            sl = pl.ds(k * 16, 16)
            gb[0, sl] = sb[0, sl] * 4 + cid
          gidx = gb.at[0]
        else:
          gidx = sb.at[0]
        pltpu.async_copy(table_h.at[gidx], rows, sem).wait()
        pltpu.sync_copy(rows, acc.at[db.at[0]], add=True)
        return 0

      lax.fori_loop(0, nb, ebody, 0, unroll=False)
      plsc.subcore_barrier()
      for r in range(SLICE // ZROWS):
        rb = rowbase + r * ZROWS
        pltpu.sync_copy(acc.at[pl.ds(rb, ZROWS)], cb)
        for k in range(n_out):
          if feature_split and (k % 2) != fc:
            continue
          @pl.when(cid == k)
          def _():
            pltpu.sync_copy(cb, outs[k].at[pl.ds(rb, ZROWS)])
      plsc.subcore_barrier()

  return agg


_agg16_edge = _make_agg16(feature_split=False)
_agg16_feat = _make_agg16(feature_split=True)


BR = 3200
_GRID = NACC // BR


def _rowspec(w):
  return pl.BlockSpec((BR, w), lambda i: (i, 0))


def _fullspec(shape):
  return pl.BlockSpec(shape, lambda i: tuple(0 for _ in shape))


def _tc_prep_kernel(d0, d1, xp, dinv, xs0):
  deg = d0[...] + d1[...] + 1.0
  di = lax.rsqrt(deg)
  dinv[...] = di
  xs0[...] = xp[...] * di


def _tc_prep(d0, d1, xp):
  return pl.pallas_call(
      _tc_prep_kernel,
      grid=(_GRID,),
      in_specs=[_rowspec(1), _rowspec(1), _rowspec(16)],
      out_specs=[_rowspec(1), _rowspec(16)],
      out_shape=[
          jax.ShapeDtypeStruct((NACC, 1), jnp.float32),
          jax.ShapeDtypeStruct((NACC, 16), jnp.float32),
      ],
  )(d0, d1, xp)


def _tc_layer1_kernel(a0, a1, xs0, dinv, w1, b1, s1):
  di = dinv[...]
  ax = di * (a0[...] + a1[...] + xs0[...])
  h1 = jnp.maximum(
      jnp.dot(ax, w1[...], preferred_element_type=jnp.float32) + b1[...], 0.0
  )
  s1[...] = h1 * di


def _tc_layer1(a0, a1, xs0, dinv, w1p, b1):
  return pl.pallas_call(
      _tc_layer1_kernel,
      grid=(_GRID,),
      in_specs=[
          _rowspec(16), _rowspec(16), _rowspec(16), _rowspec(1),
          _fullspec((16, 64)), _fullspec((1, 64)),
      ],
      out_specs=_rowspec(64),
      out_shape=jax.ShapeDtypeStruct((NACC, 64), jnp.float32),
  )(a0, a1, xs0, dinv, w1p, b1)


def _tc_layer2_kernel(c0, c1, c2, c3, s1, dinv, w2, b2, w3, zp):
  di = dinv[...]
  acc = jnp.concatenate([c0[...], c1[...], c2[...], c3[...]], axis=1)
  ah1 = di * (acc + s1[...])
  h2 = jnp.maximum(
      jnp.dot(ah1, w2[...], preferred_element_type=jnp.float32) + b2[...], 0.0
  )
  z = jnp.dot(h2, w3[...], preferred_element_type=jnp.float32) * di
  zp[...] = jnp.concatenate([z, jnp.zeros((BR, 15), jnp.float32)], axis=1)


def _tc_layer2(c0, c1, c2, c3, s1, dinv, w2, b2, w3):
  return pl.pallas_call(
      _tc_layer2_kernel,
      grid=(_GRID,),
      in_specs=[
          _rowspec(16), _rowspec(16), _rowspec(16), _rowspec(16),
          _rowspec(64), _rowspec(1),
          _fullspec((64, 64)), _fullspec((1, 64)), _fullspec((64, 1)),
      ],
      out_specs=_rowspec(16),
      out_shape=jax.ShapeDtypeStruct((NACC, 16), jnp.float32),
  )(c0, c1, c2, c3, s1, dinv, w2, b2, w3)


def _tc_final_kernel(e0, e1, zp, dinv, b3, out):
  v = dinv[...] * (e0[..., :1] + e1[..., :1] + zp[..., :1]) + b3[...]
  out[...] = v


def _tc_final(e0, e1, zp, dinv, b3):
  return pl.pallas_call(
      _tc_final_kernel,
      grid=(_GRID,),
      in_specs=[
          _rowspec(16), _rowspec(16), _rowspec(16), _rowspec(1),
          _fullspec((1, 1)),
      ],
      out_specs=_rowspec(1),
      out_shape=jax.ShapeDtypeStruct((NACC, 1), jnp.float32),
  )(e0, e1, zp, dinv, b3)


@jax.jit
def kernel(x, edge_index, W1, b1, W2, b2, W3, b3):
  pad_e = EP - E
  srcp = jnp.concatenate([edge_index[0], jnp.full((pad_e,), N, jnp.int32)])
  dstp = jnp.concatenate([edge_index[1], jnp.full((pad_e,), N, jnp.int32)])

  xp = jnp.pad(x, ((0, NACC - N), (0, 16 - x.shape[1])))
  w1p = jnp.pad(W1, ((0, 16 - W1.shape[0]), (0, 0)))

  d0, d1 = _sc_degree(dstp)
  dinv, xs0 = _tc_prep(d0[:, None], d1[:, None], xp)

  a0, a1 = _agg16_edge(xs0, srcp, dstp)
  s1 = _tc_layer1(a0, a1, xs0, dinv, w1p, b1[None, :])

  c0, c1, c2, c3 = _agg16_feat(s1.reshape(NACC * 4, 16), srcp, dstp)
  zp = _tc_layer2(c0, c1, c2, c3, s1, dinv, W2, b2[None, :], W3)

  e0, e1 = _agg16_edge(zp, srcp, dstp)
  out = _tc_final(e0, e1, zp, dinv, b3.reshape(1, 1))
  return out[:N, 0]


# speedup vs baseline: 20.9864x; 1.9856x over previous
import functools

import jax
import jax.numpy as jnp
from jax import lax
from jax.experimental import pallas as pl
from jax.experimental.pallas import tpu as pltpu
from jax.experimental.pallas import tpu_sc as plsc

N = 100000
E = 1600000
NACC = 102400
EP = 1605632
NCORE = 2
NSUB = 16
BATCH = 128
SUPB = 4
SLICE = NACC // NSUB
ZROWS = 400
NB_ES = EP // (NCORE * NSUB * BATCH)
NS_ES = NB_ES // SUPB
NB_FS = EP // (NSUB * BATCH)
NS_FS = NB_FS // SUPB

_MESH = plsc.VectorSubcoreMesh(
    core_axis_name="c", subcore_axis_name="s", num_cores=NCORE, num_subcores=NSUB
)
_SC_PARAMS = pltpu.CompilerParams(use_tc_tiling_on_sc=False)


def _zero_rows16(zb):
  def body(i, _):
    zb[i, :] = jnp.zeros((16,), jnp.float32)
    return 0
  lax.fori_loop(0, ZROWS, body, 0, unroll=False)


def _zero_acc_slice16(acc, zb, rowbase):
  for r in range(SLICE // ZROWS):
    pltpu.sync_copy(zb, acc.at[pl.ds(rowbase + r * ZROWS, ZROWS)])


@functools.partial(
    pl.kernel,
    out_type=[
        jax.ShapeDtypeStruct((NACC,), jnp.float32),
        jax.ShapeDtypeStruct((NACC,), jnp.float32),
    ],
    mesh=_MESH,
    compiler_params=_SC_PARAMS,
    scratch_types=[
        pltpu.VMEM_SHARED((NACC,), jnp.float32),
        pltpu.VMEM((SUPB, BATCH), jnp.int32),
        pltpu.VMEM((1, BATCH), jnp.float32),
        pltpu.VMEM((SLICE,), jnp.float32),
        pltpu.SemaphoreType.DMA,
    ],
)
def _sc_degree(dst2_h, out0_h, out1_h, acc, db2, ones, zb, ssem):
  c = lax.axis_index("c")
  s = lax.axis_index("s")

  def zbody(i, _):
    zb[pl.ds(i * 16, 16)] = jnp.zeros((16,), jnp.float32)
    return 0
  lax.fori_loop(0, SLICE // 16, zbody, 0, unroll=False)
  for k in range(BATCH // 16):
    ones[0, pl.ds(k * 16, 16)] = jnp.full((16,), 1.0, jnp.float32)

  rowbase = s * SLICE
  pltpu.sync_copy(zb, acc.at[pl.ds(rowbase, SLICE)])
  plsc.subcore_barrier()

  def sbody(su, _):
    row0 = (c * NSUB + s) * NB_ES + su * SUPB
    pltpu.sync_copy(dst2_h.at[pl.ds(row0, SUPB)], db2)
    descs = [
        pltpu.async_copy(ones.at[0], acc.at[db2.at[j]], ssem, add=True)
        for j in range(SUPB)
    ]
    for d in descs:
      d.wait()
    return 0
  lax.fori_loop(0, NS_ES, sbody, 0, unroll=False)

  plsc.subcore_barrier()
  pltpu.sync_copy(acc.at[pl.ds(rowbase, SLICE)], zb)
  for k, out_h in enumerate((out0_h, out1_h)):
    @pl.when(c == k)
    def _():
      pltpu.sync_copy(zb, out_h.at[pl.ds(rowbase, SLICE)])


def _make_agg16(feature_split):
  n_out = 4 if feature_split else NCORE

  @functools.partial(
      pl.kernel,
      out_type=[jax.ShapeDtypeStruct((NACC, 16), jnp.float32)] * n_out,
      mesh=_MESH,
      compiler_params=_SC_PARAMS,
      scratch_types=[
          pltpu.VMEM_SHARED((NACC, 16), jnp.float32),
          pltpu.VMEM((SUPB, BATCH), jnp.int32),
          pltpu.VMEM((SUPB, BATCH), jnp.int32),
          pltpu.VMEM((SUPB, BATCH), jnp.int32),
          pltpu.VMEM((SUPB, BATCH, 16), jnp.float32),
          pltpu.VMEM((ZROWS, 16), jnp.float32),
          pltpu.VMEM((ZROWS, 16), jnp.float32),
          pltpu.SemaphoreType.DMA,
          pltpu.SemaphoreType.DMA,
          pltpu.SemaphoreType.DMA,
      ],
  )
  def agg(table_h, src2_h, dst2_h, *rest):
    outs = rest[:n_out]
    acc, sb2, db2, gb2, rows, zb, cb, gsem0, gsem1, ssem = rest[n_out:]
    gsems = (gsem0, gsem1)
    c = lax.axis_index("c")
    s = lax.axis_index("s")
    _zero_rows16(zb)
    rowbase = s * SLICE

    for fc in range(2 if feature_split else 1):
      if feature_split:
        cid = c * 2 + fc
        ns = NS_FS
      else:
        cid = c
        ns = NS_ES

      _zero_acc_slice16(acc, zb, rowbase)
      plsc.subcore_barrier()

      def sbody(su, _):
        if feature_split:
          row0 = s * NB_FS + su * SUPB
        else:
          row0 = (c * NSUB + s) * NB_ES + su * SUPB
        pltpu.sync_copy(src2_h.at[pl.ds(row0, SUPB)], sb2)
        pltpu.sync_copy(dst2_h.at[pl.ds(row0, SUPB)], db2)
        if feature_split:
          for j in range(SUPB):
            for k in range(BATCH // 16):
              sl = pl.ds(k * 16, 16)
              gb2[j, sl] = sb2[j, sl] * 4 + cid
          gref = gb2
        else:
          gref = sb2
        gd = {}
        for j in range(2):
          gd[j] = pltpu.async_copy(table_h.at[gref.at[j]], rows.at[j], gsems[j])
        sds = []
        for j in range(SUPB):
          gd[j].wait()
          sds.append(
              pltpu.async_copy(rows.at[j], acc.at[db2.at[j]], ssem, add=True)
          )
          if j + 2 < SUPB:
            gd[j + 2] = pltpu.async_copy(
                table_h.at[gref.at[j + 2]], rows.at[j + 2], gsems[j % 2]
            )
        for d in sds:
          d.wait()
        return 0

      lax.fori_loop(0, ns, sbody, 0, unroll=False)
      plsc.subcore_barrier()
      for r in range(SLICE // ZROWS):
        rb = rowbase + r * ZROWS
        pltpu.sync_copy(acc.at[pl.ds(rb, ZROWS)], cb)
        for k in range(n_out):
          if feature_split and (k % 2) != fc:
            continue
          @pl.when(cid == k)
          def _():
            pltpu.sync_copy(cb, outs[k].at[pl.ds(rb, ZROWS)])
      plsc.subcore_barrier()

  return agg


_agg16_edge = _make_agg16(feature_split=False)
_agg16_feat = _make_agg16(feature_split=True)


BR = 3200
_GRID = NACC // BR


def _rowspec(w):
  return pl.BlockSpec((BR, w), lambda i: (i, 0))


def _fullspec(shape):
  return pl.BlockSpec(shape, lambda i: tuple(0 for _ in shape))


def _tc_prep_kernel(d0, d1, xp, dinv, xs0):
  deg = d0[...] + d1[...] + 1.0
  di = lax.rsqrt(deg)
  dinv[...] = di
  xs0[...] = xp[...] * di


def _tc_prep(d0, d1, xp):
  return pl.pallas_call(
      _tc_prep_kernel,
      grid=(_GRID,),
      in_specs=[_rowspec(1), _rowspec(1), _rowspec(16)],
      out_specs=[_rowspec(1), _rowspec(16)],
      out_shape=[
          jax.ShapeDtypeStruct((NACC, 1), jnp.float32),
          jax.ShapeDtypeStruct((NACC, 16), jnp.float32),
      ],
  )(d0, d1, xp)


def _tc_layer1_kernel(a0, a1, xs0, dinv, w1, b1, s1):
  di = dinv[...]
  ax = di * (a0[...] + a1[...] + xs0[...])
  h1 = jnp.maximum(
      jnp.dot(ax, w1[...], preferred_element_type=jnp.float32) + b1[...], 0.0
  )
  s1[...] = h1 * di


def _tc_layer1(a0, a1, xs0, dinv, w1p, b1):
  return pl.pallas_call(
      _tc_layer1_kernel,
      grid=(_GRID,),
      in_specs=[
          _rowspec(16), _rowspec(16), _rowspec(16), _rowspec(1),
          _fullspec((16, 64)), _fullspec((1, 64)),
      ],
      out_specs=_rowspec(64),
      out_shape=jax.ShapeDtypeStruct((NACC, 64), jnp.float32),
  )(a0, a1, xs0, dinv, w1p, b1)


def _tc_layer2_kernel(c0, c1, c2, c3, s1, dinv, w2, b2, w3, zp):
  di = dinv[...]
  acc = jnp.concatenate([c0[...], c1[...], c2[...], c3[...]], axis=1)
  ah1 = di * (acc + s1[...])
  h2 = jnp.maximum(
      jnp.dot(ah1, w2[...], preferred_element_type=jnp.float32) + b2[...], 0.0
  )
  z = jnp.dot(h2, w3[...], preferred_element_type=jnp.float32) * di
  zp[...] = jnp.concatenate([z, jnp.zeros((BR, 15), jnp.float32)], axis=1)


def _tc_layer2(c0, c1, c2, c3, s1, dinv, w2, b2, w3):
  return pl.pallas_call(
      _tc_layer2_kernel,
      grid=(_GRID,),
      in_specs=[
          _rowspec(16), _rowspec(16), _rowspec(16), _rowspec(16),
          _rowspec(64), _rowspec(1),
          _fullspec((64, 64)), _fullspec((1, 64)), _fullspec((64, 1)),
      ],
      out_specs=_rowspec(16),
      out_shape=jax.ShapeDtypeStruct((NACC, 16), jnp.float32),
  )(c0, c1, c2, c3, s1, dinv, w2, b2, w3)


def _tc_final_kernel(e0, e1, zp, dinv, b3, out):
  v = dinv[...] * (e0[..., :1] + e1[..., :1] + zp[..., :1]) + b3[...]
  out[...] = v


def _tc_final(e0, e1, zp, dinv, b3):
  return pl.pallas_call(
      _tc_final_kernel,
      grid=(_GRID,),
      in_specs=[
          _rowspec(16), _rowspec(16), _rowspec(16), _rowspec(1),
          _fullspec((1, 1)),
      ],
      out_specs=_rowspec(1),
      out_shape=jax.ShapeDtypeStruct((NACC, 1), jnp.float32),
  )(e0, e1, zp, dinv, b3)


@jax.jit
def kernel(x, edge_index, W1, b1, W2, b2, W3, b3):
  pad_e = EP - E
  srcp = jnp.concatenate(
      [edge_index[0], jnp.full((pad_e,), N, jnp.int32)]
  ).reshape(EP // BATCH, BATCH)
  dstp = jnp.concatenate(
      [edge_index[1], jnp.full((pad_e,), N, jnp.int32)]
  ).reshape(EP // BATCH, BATCH)

  xp = jnp.pad(x, ((0, NACC - N), (0, 16 - x.shape[1])))
  w1p = jnp.pad(W1, ((0, 16 - W1.shape[0]), (0, 0)))

  d0, d1 = _sc_degree(dstp)
  dinv, xs0 = _tc_prep(d0[:, None], d1[:, None], xp)

  a0, a1 = _agg16_edge(xs0, srcp, dstp)
  s1 = _tc_layer1(a0, a1, xs0, dinv, w1p, b1[None, :])

  c0, c1, c2, c3 = _agg16_feat(s1.reshape(NACC * 4, 16), srcp, dstp)
  zp = _tc_layer2(c0, c1, c2, c3, s1, dinv, W2, b2[None, :], W3)

  e0, e1 = _agg16_edge(zp, srcp, dstp)
  out = _tc_final(e0, e1, zp, dinv, b3.reshape(1, 1))
  return out[:N, 0]


# speedup vs baseline: 25.3056x; 1.2058x over previous
import functools

import jax
import jax.numpy as jnp
from jax import lax
from jax.experimental import pallas as pl
from jax.experimental.pallas import tpu as pltpu
from jax.experimental.pallas import tpu_sc as plsc

N = 100000
E = 1600000
NACC = 102400
EP = 1605632
NCORE = 2
NSUB = 16
BATCH = 128
SUPB = 8
SLICE = NACC // NSUB
ZROWS = 200
NB_ES = EP // (NCORE * NSUB * BATCH)
NS_ES = NB_ES // SUPB
NB_FS = EP // (NSUB * BATCH)
NS_FS = NB_FS // SUPB

_MESH = plsc.VectorSubcoreMesh(
    core_axis_name="c", subcore_axis_name="s", num_cores=NCORE, num_subcores=NSUB
)
_SC_PARAMS = pltpu.CompilerParams(use_tc_tiling_on_sc=False)


def _zero_rows16(zb):
  def body(i, _):
    zb[i, :] = jnp.zeros((16,), jnp.float32)
    return 0
  lax.fori_loop(0, ZROWS, body, 0, unroll=False)


def _zero_acc_slice16(acc, zb, rowbase):
  for r in range(SLICE // ZROWS):
    pltpu.sync_copy(zb, acc.at[pl.ds(rowbase + r * ZROWS, ZROWS)])


@functools.partial(
    pl.kernel,
    out_type=[
        jax.ShapeDtypeStruct((NACC,), jnp.float32),
        jax.ShapeDtypeStruct((NACC,), jnp.float32),
    ],
    mesh=_MESH,
    compiler_params=_SC_PARAMS,
    scratch_types=[
        pltpu.VMEM_SHARED((NACC,), jnp.float32),
        pltpu.VMEM((SUPB, BATCH), jnp.int32),
        pltpu.VMEM((1, BATCH), jnp.float32),
        pltpu.VMEM((SLICE,), jnp.float32),
        pltpu.SemaphoreType.DMA,
    ],
)
def _sc_degree(dst2_h, out0_h, out1_h, acc, db2, ones, zb, ssem):
  c = lax.axis_index("c")
  s = lax.axis_index("s")

  def zbody(i, _):
    zb[pl.ds(i * 16, 16)] = jnp.zeros((16,), jnp.float32)
    return 0
  lax.fori_loop(0, SLICE // 16, zbody, 0, unroll=False)
  for k in range(BATCH // 16):
    ones[0, pl.ds(k * 16, 16)] = jnp.full((16,), 1.0, jnp.float32)

  rowbase = s * SLICE
  pltpu.sync_copy(zb, acc.at[pl.ds(rowbase, SLICE)])
  plsc.subcore_barrier()

  def sbody(su, _):
    row0 = (c * NSUB + s) * NB_ES + su * SUPB
    pltpu.sync_copy(dst2_h.at[pl.ds(row0, SUPB)], db2)
    descs = [
        pltpu.async_copy(ones.at[0], acc.at[db2.at[j]], ssem, add=True)
        for j in range(SUPB)
    ]
    for d in descs:
      d.wait()
    return 0
  lax.fori_loop(0, NS_ES, sbody, 0, unroll=False)

  plsc.subcore_barrier()
  pltpu.sync_copy(acc.at[pl.ds(rowbase, SLICE)], zb)
  for k, out_h in enumerate((out0_h, out1_h)):
    @pl.when(c == k)
    def _():
      pltpu.sync_copy(zb, out_h.at[pl.ds(rowbase, SLICE)])


def _make_agg16(feature_split):
  n_out = 4 if feature_split else NCORE

  @functools.partial(
      pl.kernel,
      out_type=[jax.ShapeDtypeStruct((NACC, 16), jnp.float32)] * n_out,
      mesh=_MESH,
      compiler_params=_SC_PARAMS,
      scratch_types=[
          pltpu.VMEM_SHARED((NACC, 16), jnp.float32),
          pltpu.VMEM((SUPB, BATCH), jnp.int32),
          pltpu.VMEM((SUPB, BATCH), jnp.int32),
          pltpu.VMEM((SUPB, BATCH), jnp.int32),
          pltpu.VMEM((SUPB, BATCH, 16), jnp.float32),
          pltpu.VMEM((ZROWS, 16), jnp.float32),
          pltpu.VMEM((ZROWS, 16), jnp.float32),
          pltpu.SemaphoreType.DMA,
          pltpu.SemaphoreType.DMA,
          pltpu.SemaphoreType.DMA,
          pltpu.SemaphoreType.DMA,
          pltpu.SemaphoreType.DMA,
      ],
  )
  def agg(table_h, src2_h, dst2_h, *rest):
    outs = rest[:n_out]
    acc, sb2, db2, gb2, rows, zb, cb, gsem0, gsem1, ssem, isem0, isem1 = rest[n_out:]
    gsems = (gsem0, gsem1)
    c = lax.axis_index("c")
    s = lax.axis_index("s")
    _zero_rows16(zb)
    rowbase = s * SLICE

    for fc in range(2 if feature_split else 1):
      if feature_split:
        cid = c * 2 + fc
        ns = NS_FS
      else:
        cid = c
        ns = NS_ES

      _zero_acc_slice16(acc, zb, rowbase)
      plsc.subcore_barrier()

      def sbody(su, _):
        if feature_split:
          row0 = s * NB_FS + su * SUPB
        else:
          row0 = (c * NSUB + s) * NB_ES + su * SUPB
        i0 = pltpu.async_copy(src2_h.at[pl.ds(row0, SUPB)], sb2, isem0)
        i1 = pltpu.async_copy(dst2_h.at[pl.ds(row0, SUPB)], db2, isem1)
        i0.wait()
        i1.wait()
        if feature_split:
          for j in range(SUPB):
            for k in range(BATCH // 16):
              sl = pl.ds(k * 16, 16)
              gb2[j, sl] = sb2[j, sl] * 4 + cid
          gref = gb2
        else:
          gref = sb2
        gd = {}
        for j in range(2):
          gd[j] = pltpu.async_copy(table_h.at[gref.at[j]], rows.at[j], gsems[j])
        sds = []
        for j in range(SUPB):
          gd[j].wait()
          sds.append(
              pltpu.async_copy(rows.at[j], acc.at[db2.at[j]], ssem, add=True)
          )
          if j + 2 < SUPB:
            gd[j + 2] = pltpu.async_copy(
                table_h.at[gref.at[j + 2]], rows.at[j + 2], gsems[j % 2]
            )
        for d in sds:
          d.wait()
        return 0

      lax.fori_loop(0, ns, sbody, 0, unroll=False)
      plsc.subcore_barrier()
      for r in range(SLICE // ZROWS):
        rb = rowbase + r * ZROWS
        pltpu.sync_copy(acc.at[pl.ds(rb, ZROWS)], cb)
        for k in range(n_out):
          if feature_split and (k % 2) != fc:
            continue
          @pl.when(cid == k)
          def _():
            pltpu.sync_copy(cb, outs[k].at[pl.ds(rb, ZROWS)])
      plsc.subcore_barrier()

  return agg


_agg16_edge = _make_agg16(feature_split=False)
_agg16_feat = _make_agg16(feature_split=True)


BR = 3200
_GRID = NACC // BR


def _rowspec(w):
  return pl.BlockSpec((BR, w), lambda i: (i, 0))


def _fullspec(shape):
  return pl.BlockSpec(shape, lambda i: tuple(0 for _ in shape))


def _tc_prep_kernel(d0, d1, xp, dinv, xs0):
  deg = d0[...] + d1[...] + 1.0
  di = lax.rsqrt(deg)
  dinv[...] = di
  xs0[...] = xp[...] * di


def _tc_prep(d0, d1, xp):
  return pl.pallas_call(
      _tc_prep_kernel,
      grid=(_GRID,),
      in_specs=[_rowspec(1), _rowspec(1), _rowspec(16)],
      out_specs=[_rowspec(1), _rowspec(16)],
      out_shape=[
          jax.ShapeDtypeStruct((NACC, 1), jnp.float32),
          jax.ShapeDtypeStruct((NACC, 16), jnp.float32),
      ],
  )(d0, d1, xp)


def _tc_layer1_kernel(a0, a1, xs0, dinv, w1, b1, s1):
  di = dinv[...]
  ax = di * (a0[...] + a1[...] + xs0[...])
  h1 = jnp.maximum(
      jnp.dot(ax, w1[...], preferred_element_type=jnp.float32) + b1[...], 0.0
  )
  s1[...] = h1 * di


def _tc_layer1(a0, a1, xs0, dinv, w1p, b1):
  return pl.pallas_call(
      _tc_layer1_kernel,
      grid=(_GRID,),
      in_specs=[
          _rowspec(16), _rowspec(16), _rowspec(16), _rowspec(1),
          _fullspec((16, 64)), _fullspec((1, 64)),
      ],
      out_specs=_rowspec(64),
      out_shape=jax.ShapeDtypeStruct((NACC, 64), jnp.float32),
  )(a0, a1, xs0, dinv, w1p, b1)


def _tc_layer2_kernel(c0, c1, c2, c3, s1, dinv, w2, b2, w3, zp):
  di = dinv[...]
  acc = jnp.concatenate([c0[...], c1[...], c2[...], c3[...]], axis=1)
  ah1 = di * (acc + s1[...])
  h2 = jnp.maximum(
      jnp.dot(ah1, w2[...], preferred_element_type=jnp.float32) + b2[...], 0.0
  )
  z = jnp.dot(h2, w3[...], preferred_element_type=jnp.float32) * di
  zp[...] = jnp.concatenate([z, jnp.zeros((BR, 15), jnp.float32)], axis=1)


def _tc_layer2(c0, c1, c2, c3, s1, dinv, w2, b2, w3):
  return pl.pallas_call(
      _tc_layer2_kernel,
      grid=(_GRID,),
      in_specs=[
          _rowspec(16), _rowspec(16), _rowspec(16), _rowspec(16),
          _rowspec(64), _rowspec(1),
          _fullspec((64, 64)), _fullspec((1, 64)), _fullspec((64, 1)),
      ],
      out_specs=_rowspec(16),
      out_shape=jax.ShapeDtypeStruct((NACC, 16), jnp.float32),
  )(c0, c1, c2, c3, s1, dinv, w2, b2, w3)


def _tc_final_kernel(e0, e1, zp, dinv, b3, out):
  v = dinv[...] * (e0[..., :1] + e1[..., :1] + zp[..., :1]) + b3[...]
  out[...] = v


def _tc_final(e0, e1, zp, dinv, b3):
  return pl.pallas_call(
      _tc_final_kernel,
      grid=(_GRID,),
      in_specs=[
          _rowspec(16), _rowspec(16), _rowspec(16), _rowspec(1),
          _fullspec((1, 1)),
      ],
      out_specs=_rowspec(1),
      out_shape=jax.ShapeDtypeStruct((NACC, 1), jnp.float32),
  )(e0, e1, zp, dinv, b3)


@jax.jit
def kernel(x, edge_index, W1, b1, W2, b2, W3, b3):
  pad_e = EP - E
  srcp = jnp.concatenate(
      [edge_index[0], jnp.full((pad_e,), N, jnp.int32)]
  ).reshape(EP // BATCH, BATCH)
  dstp = jnp.concatenate(
      [edge_index[1], jnp.full((pad_e,), N, jnp.int32)]
  ).reshape(EP // BATCH, BATCH)

  xp = jnp.pad(x, ((0, NACC - N), (0, 16 - x.shape[1])))
  w1p = jnp.pad(W1, ((0, 16 - W1.shape[0]), (0, 0)))

  d0, d1 = _sc_degree(dstp)
  dinv, xs0 = _tc_prep(d0[:, None], d1[:, None], xp)

  a0, a1 = _agg16_edge(xs0, srcp, dstp)
  s1 = _tc_layer1(a0, a1, xs0, dinv, w1p, b1[None, :])

  c0, c1, c2, c3 = _agg16_feat(s1.reshape(NACC * 4, 16), srcp, dstp)
  zp = _tc_layer2(c0, c1, c2, c3, s1, dinv, W2, b2[None, :], W3)

  e0, e1 = _agg16_edge(zp, srcp, dstp)
  out = _tc_final(e0, e1, zp, dinv, b3.reshape(1, 1))
  return out[:N, 0]
